# baseline (speedup 1.0000x reference)
import functools
import jax
import jax.numpy as jnp
from jax.experimental import pallas as pl
from jax.experimental.pallas import tpu as pltpu

_B = 16
_D = 64


def _eye(n, dtype=jnp.float32):
    return (jax.lax.broadcasted_iota(jnp.int32, (n, n), 0)
            == jax.lax.broadcasted_iota(jnp.int32, (n, n), 1)).astype(dtype)


def _fused_body(nb, seg_ref, xT_ref, fu_ref, wu_ref, bu_ref, wk_ref,
                we_ref, out_ref, uT_scr, wkh_scr, weT_scr, m_scr,
                s_scr, acc_scr):
    i = pl.program_id(0)
    C = xT_ref.shape[1]

    @pl.when(i == 0)
    def _init():
        eye16 = _eye(_B)
        eye64 = _eye(_D)
        u = jnp.dot(fu_ref[...], wu_ref[...],
                    preferred_element_type=jnp.float32) + bu_ref[...]
        uT_scr[...] = 0.5 * jax.lax.dot_general(
            u, eye16, (((0,), (0,)), ((), ())),
            preferred_element_type=jnp.float32)
        wkh_scr[...] = 0.5 * jax.lax.dot_general(
            wk_ref[...], eye64, (((0,), (0,)), ((), ())),
            preferred_element_type=jnp.float32)
        weT = jnp.sum(eye64 * we_ref[...], axis=0, keepdims=True)
        weT_scr[...] = weT
        we_sum = jnp.sum(weT, axis=1, keepdims=True)
        m_bound = jnp.sum(jnp.maximum(weT, 0.0), axis=1, keepdims=True)
        m_scr[0:1, 0:1] = m_bound
        m_scr[1:2, 0:1] = we_sum
        s_scr[...] = jnp.zeros(s_scr.shape, s_scr.dtype)
        acc_scr[...] = jnp.zeros(acc_scr.shape, acc_scr.dtype)

    xT = xT_ref[...]
    seg_row = seg_ref[0]
    onehot_b = seg_row == jax.lax.broadcasted_iota(jnp.int32, (_B, C), 0)
    onehot = onehot_b.astype(jnp.float32)

    keyT = jnp.dot(wkh_scr[...], xT, preferred_element_type=jnp.float32)
    qryT = jnp.dot(uT_scr[...], onehot, preferred_element_type=jnp.float32)
    thT = jnp.tanh(keyT + qryT)
    e2_row = (jnp.dot(weT_scr[...], thT, preferred_element_type=jnp.float32)
              + m_scr[1:2, 0:1])
    p_row = jnp.exp(0.5 * e2_row - m_scr[0:1, 0:1])
    whT = onehot * p_row
    s_new = s_scr[...] + jnp.sum(whT, axis=1, keepdims=True)
    acc_new = acc_scr[...] + jax.lax.dot_general(
        whT, xT, (((1,), (1,)), ((), ())),
        preferred_element_type=jnp.float32)
    s_scr[...] = s_new
    acc_scr[...] = acc_new

    @pl.when(i == nb - 1)
    def _fin():
        out_ref[...] = jnp.where(s_new > 0, acc_new / s_new, 0.0)


def kernel(feat_i, feat_u, segment_ids, W_user, b_user, W_key, W_e):
    N, D = feat_i.shape
    C = 8192
    nb = N // C
    xT = feat_i.T
    bu = b_user.reshape(1, D)
    seg3 = segment_ids.reshape(nb, 1, C)

    return pl.pallas_call(
        functools.partial(_fused_body, nb),
        grid=(nb,),
        in_specs=[
            pl.BlockSpec((1, 1, C), lambda i: (i, 0, 0)),
            pl.BlockSpec((D, C), lambda i: (0, i)),
            pl.BlockSpec((_B, D), lambda i: (0, 0)),
            pl.BlockSpec((D, D), lambda i: (0, 0)),
            pl.BlockSpec((1, D), lambda i: (0, 0)),
            pl.BlockSpec((D, D), lambda i: (0, 0)),
            pl.BlockSpec((D, 1), lambda i: (0, 0)),
        ],
        out_specs=pl.BlockSpec((_B, D), lambda i: (0, 0)),
        out_shape=jax.ShapeDtypeStruct((_B, D), jnp.float32),
        scratch_shapes=[
            pltpu.VMEM((_D, _B), jnp.float32),
            pltpu.VMEM((_D, _D), jnp.float32),
            pltpu.VMEM((1, _D), jnp.float32),
            pltpu.VMEM((2, 1), jnp.float32),
            pltpu.VMEM((_B, 1), jnp.float32),
            pltpu.VMEM((_B, _D), jnp.float32),
        ],
        compiler_params=pltpu.CompilerParams(
            dimension_semantics=("arbitrary",),
        ),
    )(seg3, xT, feat_u, W_user, bu, W_key, W_e)


# speedup vs baseline: 1.0082x; 1.0082x over previous
import functools
import jax
import jax.numpy as jnp
from jax.experimental import pallas as pl
from jax.experimental.pallas import tpu as pltpu

_B = 16
_D = 64


def _eye(n, dtype=jnp.float32):
    return (jax.lax.broadcasted_iota(jnp.int32, (n, n), 0)
            == jax.lax.broadcasted_iota(jnp.int32, (n, n), 1)).astype(dtype)


def _fused_body(nb, seg_ref, xT_ref, fu_ref, wu_ref, bu_ref, wk_ref,
                we_ref, out_ref, uT_scr, wkh_scr, weT_scr, m_scr,
                s_scr, acc_scr):
    i = pl.program_id(0)
    C = xT_ref.shape[1]

    @pl.when(i == 0)
    def _init():
        eye16 = _eye(_B)
        eye64 = _eye(_D)
        u = jnp.dot(fu_ref[...], wu_ref[...],
                    preferred_element_type=jnp.float32) + bu_ref[...]
        uT_scr[...] = 0.5 * jax.lax.dot_general(
            u, eye16, (((0,), (0,)), ((), ())),
            preferred_element_type=jnp.float32)
        wkh_scr[...] = 0.5 * jax.lax.dot_general(
            wk_ref[...], eye64, (((0,), (0,)), ((), ())),
            preferred_element_type=jnp.float32)
        weT = jnp.sum(eye64 * we_ref[...], axis=0, keepdims=True)
        weT_scr[...] = weT
        we_sum = jnp.sum(weT, axis=1, keepdims=True)
        m_bound = jnp.sum(jnp.maximum(weT, 0.0), axis=1, keepdims=True)
        m_scr[0:1, 0:1] = m_bound
        m_scr[1:2, 0:1] = we_sum
        s_scr[...] = jnp.zeros(s_scr.shape, s_scr.dtype)
        acc_scr[...] = jnp.zeros(acc_scr.shape, acc_scr.dtype)

    xT = xT_ref[...]
    seg_row = seg_ref[0]
    onehot_b = seg_row == jax.lax.broadcasted_iota(jnp.int32, (_B, C), 0)
    onehot = onehot_b.astype(jnp.float32)

    keyT = jnp.dot(wkh_scr[...], xT, preferred_element_type=jnp.float32)
    qryT = jnp.dot(uT_scr[...], onehot, preferred_element_type=jnp.float32)
    thT = jnp.tanh(keyT + qryT)
    e2_row = (jnp.dot(weT_scr[...], thT, preferred_element_type=jnp.float32)
              + m_scr[1:2, 0:1])
    p_row = jnp.exp(0.5 * e2_row - m_scr[0:1, 0:1])
    whT = onehot * p_row
    s_new = s_scr[...] + jnp.sum(whT, axis=1, keepdims=True)
    acc_new = acc_scr[...] + jax.lax.dot_general(
        whT, xT, (((1,), (1,)), ((), ())),
        preferred_element_type=jnp.float32)
    s_scr[...] = s_new
    acc_scr[...] = acc_new

    @pl.when(i == nb - 1)
    def _fin():
        out_ref[...] = jnp.where(s_new > 0, acc_new / s_new, 0.0)


def kernel(feat_i, feat_u, segment_ids, W_user, b_user, W_key, W_e):
    N, D = feat_i.shape
    C = 16384
    nb = N // C
    xT = feat_i.T
    bu = b_user.reshape(1, D)
    seg3 = segment_ids.reshape(nb, 1, C)

    return pl.pallas_call(
        functools.partial(_fused_body, nb),
        grid=(nb,),
        in_specs=[
            pl.BlockSpec((1, 1, C), lambda i: (i, 0, 0)),
            pl.BlockSpec((D, C), lambda i: (0, i)),
            pl.BlockSpec((_B, D), lambda i: (0, 0)),
            pl.BlockSpec((D, D), lambda i: (0, 0)),
            pl.BlockSpec((1, D), lambda i: (0, 0)),
            pl.BlockSpec((D, D), lambda i: (0, 0)),
            pl.BlockSpec((D, 1), lambda i: (0, 0)),
        ],
        out_specs=pl.BlockSpec((_B, D), lambda i: (0, 0)),
        out_shape=jax.ShapeDtypeStruct((_B, D), jnp.float32),
        scratch_shapes=[
            pltpu.VMEM((_D, _B), jnp.float32),
            pltpu.VMEM((_D, _D), jnp.float32),
            pltpu.VMEM((1, _D), jnp.float32),
            pltpu.VMEM((2, 1), jnp.float32),
            pltpu.VMEM((_B, 1), jnp.float32),
            pltpu.VMEM((_B, _D), jnp.float32),
        ],
        compiler_params=pltpu.CompilerParams(
            dimension_semantics=("arbitrary",),
        ),
    )(seg3, xT, feat_u, W_user, bu, W_key, W_e)
